# Initial kernel scaffold; baseline (speedup 1.0000x reference)
import jax
import jax.numpy as jnp
from jax.experimental import pallas as pl


def kernel(node_embeddings, hyperedge_embeddings, hyperedge_to_nodes, W_fc, b_fc, W_we, b_we, c_e):
    raise NotImplementedError("write your pallas kernel here")


# speedup vs baseline: 4.0015x; 4.0015x over previous
import functools

import jax
import jax.numpy as jnp
from jax import lax
from jax.experimental import pallas as pl
from jax.experimental.pallas import tpu as pltpu
from jax.experimental.pallas import tpu_sc as plsc


def _edge_kernel(he_ref, wt_ref, b_ref, nodes_ref, t0_ref, t1_ref, t2_ref, t3_ref, cc_ref):
    ht = jnp.dot(he_ref[...], wt_ref[...], preferred_element_type=jnp.float32) + b_ref[...]
    t0_ref[...] = ht[:, 0:128]
    t1_ref[...] = ht[:, 128:256]
    t2_ref[...] = ht[:, 256:384]
    t3_ref[...] = ht[:, 384:512]
    n = nodes_ref[...]
    rows, K = n.shape
    dup = jnp.zeros(n.shape, jnp.bool_)
    for d in range(1, K):
        shifted = jnp.concatenate(
            [jnp.full((rows, d), -1, jnp.int32), n[:, : K - d]], axis=1)
        dup = jnp.logical_or(dup, n == shifted)
    cc_ref[...] = 1.0 - dup.astype(jnp.float32)


def _node_kernel(x_ref, wt_ref, b_ref, ce_ref, s_ref):
    t = jnp.dot(x_ref[...], wt_ref[...], preferred_element_type=jnp.float32) + b_ref[...]
    lr = jnp.where(t >= 0, t, 0.01 * t)
    s_ref[...] = jnp.dot(lr, ce_ref[...], preferred_element_type=jnp.float32)


def _combine_kernel(H, S_ref, s_ref, cnt_ref, o_ref):
    s = s_ref[...]
    c = cnt_ref[...]
    m = jnp.maximum(s, 0.0)
    e = jnp.maximum(jnp.exp(s - m), 1e-35)
    denom = c * e + (float(H) - c) * jnp.exp(-m)
    a = e / denom
    o_ref[...] = jnp.maximum(a * S_ref[0], 0.0)


def _make_sc_scatter(NPAD, n_tiles, blocks_per_tile, stripe):
    mesh = plsc.VectorSubcoreMesh(core_axis_name="c", subcore_axis_name="s")

    @functools.partial(
        pl.kernel,
        mesh=mesh,
        out_type=[
            jax.ShapeDtypeStruct((4, NPAD, 128), jnp.float32),
            jax.ShapeDtypeStruct((NPAD,), jnp.float32),
        ],
        scratch_types=[
            pltpu.VMEM((blocks_per_tile, 128), jnp.int32),
            pltpu.VMEM((blocks_per_tile, 128), jnp.int32),
            pltpu.VMEM((blocks_per_tile, 128), jnp.float32),
            pltpu.VMEM((128, 128), jnp.float32),
            pltpu.VMEM_SHARED((NPAD, 128), jnp.float32),
            pltpu.VMEM_SHARED((NPAD,), jnp.float32),
            pltpu.SemaphoreType.DMA,
        ],
    )
    def sc_scatter(nodes3, eids3, cntv3, zrows, z1, t0, t1, t2, t3,
                   S_out, cnt_out,
                   idxn_v, idxe_v, cval_v, rows_v, S_sh, cnt_sh, sem):
        cid = lax.axis_index("c")
        sid = lax.axis_index("s")

        pltpu.sync_copy(nodes3.at[sid], idxn_v)
        pltpu.sync_copy(eids3.at[sid], idxe_v)

        @pl.when(cid == 0)
        def _():
            pltpu.sync_copy(z1, cnt_sh.at[pl.ds(sid * stripe, stripe)])
            plsc.subcore_barrier()
            pltpu.sync_copy(cntv3.at[sid], cval_v)

            def cbody(j, carry):
                pltpu.sync_copy(cval_v.at[j], cnt_sh.at[idxn_v.at[j]], add=True)
                return carry

            lax.fori_loop(0, blocks_per_tile, cbody, 0)
            plsc.subcore_barrier()
            pltpu.sync_copy(cnt_sh.at[pl.ds(sid * stripe, stripe)],
                            cnt_out.at[pl.ds(sid * stripe, stripe)])

        for ti, table in enumerate((t0, t1, t2, t3)):
            @pl.when(cid == ti // 2)
            def _(table=table, ti=ti):
                pltpu.sync_copy(zrows, S_sh.at[pl.ds(sid * stripe, stripe)])
                plsc.subcore_barrier()

                def body(j, carry):
                    pltpu.async_copy(table.at[idxe_v.at[j]], rows_v, sem).wait()
                    pltpu.sync_copy(rows_v, S_sh.at[idxn_v.at[j]], add=True)
                    return carry

                lax.fori_loop(0, blocks_per_tile, body, 0)
                plsc.subcore_barrier()
                pltpu.sync_copy(S_sh.at[pl.ds(sid * stripe, stripe)],
                                S_out.at[ti, pl.ds(sid * stripe, stripe)])

    return sc_scatter


def kernel(node_embeddings, hyperedge_embeddings, hyperedge_to_nodes, W_fc, b_fc, W_we, b_we, c_e):
    N, d_in = node_embeddings.shape
    H, K = hyperedge_to_nodes.shape
    d_out = W_fc.shape[0]
    assert d_in == 512 and d_out == 512

    n_tiles = 16
    stripe = 640
    NPAD = n_tiles * stripe
    PAIRS = H * K
    pairs_per_tile = PAIRS // n_tiles
    blocks_per_tile = pairs_per_tile // 128

    f32 = jnp.float32

    eb = 256
    ht0, ht1, ht2, ht3, cc = pl.pallas_call(
        _edge_kernel,
        grid=(H // eb,),
        in_specs=[
            pl.BlockSpec((eb, d_in), lambda i: (i, 0)),
            pl.BlockSpec((d_in, d_out), lambda i: (0, 0)),
            pl.BlockSpec((1, d_out), lambda i: (0, 0)),
            pl.BlockSpec((eb, K), lambda i: (i, 0)),
        ],
        out_specs=[pl.BlockSpec((eb, 128), lambda i: (i, 0))] * 4
        + [pl.BlockSpec((eb, K), lambda i: (i, 0))],
        out_shape=[jax.ShapeDtypeStruct((H, 128), f32)] * 4
        + [jax.ShapeDtypeStruct((H, K), f32)],
    )(hyperedge_embeddings, W_fc.T, b_fc.reshape(1, -1), hyperedge_to_nodes)

    nb = 640
    ne_pad = jnp.pad(node_embeddings, ((0, NPAD - N), (0, 0)))
    s_col = pl.pallas_call(
        _node_kernel,
        grid=(NPAD // nb,),
        in_specs=[
            pl.BlockSpec((nb, d_in), lambda i: (i, 0)),
            pl.BlockSpec((d_in, d_out), lambda i: (0, 0)),
            pl.BlockSpec((1, d_out), lambda i: (0, 0)),
            pl.BlockSpec((d_out, 1), lambda i: (0, 0)),
        ],
        out_specs=pl.BlockSpec((nb, 1), lambda i: (i, 0)),
        out_shape=jax.ShapeDtypeStruct((NPAD, 1), f32),
    )(ne_pad, W_we.T, b_we.reshape(1, -1), c_e.reshape(-1, 1))

    nodes3 = hyperedge_to_nodes.reshape(n_tiles, blocks_per_tile, 128)
    eids3 = (jnp.arange(PAIRS, dtype=jnp.int32) // K).reshape(
        n_tiles, blocks_per_tile, 128)
    cntv3 = cc.reshape(n_tiles, blocks_per_tile, 128)
    zrows = jnp.zeros((stripe, 128), f32)
    z1 = jnp.zeros((stripe,), f32)

    sc = _make_sc_scatter(NPAD, n_tiles, blocks_per_tile, stripe)
    S_chunks, cnt = sc(nodes3, eids3, cntv3, zrows, z1, ht0, ht1, ht2, ht3)

    out_pad = pl.pallas_call(
        functools.partial(_combine_kernel, H),
        grid=(NPAD // nb, 4),
        in_specs=[
            pl.BlockSpec((1, nb, 128), lambda r, c: (c, r, 0)),
            pl.BlockSpec((nb, 1), lambda r, c: (r, 0)),
            pl.BlockSpec((nb, 1), lambda r, c: (r, 0)),
        ],
        out_specs=pl.BlockSpec((nb, 128), lambda r, c: (r, c)),
        out_shape=jax.ShapeDtypeStruct((NPAD, d_out), f32),
    )(S_chunks, s_col, cnt.reshape(NPAD, 1))

    return out_pad[:N]


# speedup vs baseline: 5.2023x; 1.3001x over previous
import functools

import jax
import jax.numpy as jnp
from jax import lax
from jax.experimental import pallas as pl
from jax.experimental.pallas import tpu as pltpu
from jax.experimental.pallas import tpu_sc as plsc


def _edge_kernel(he_ref, wt_ref, b_ref, nodes_ref, t0_ref, t1_ref, t2_ref, t3_ref, cc_ref):
    ht = jnp.dot(he_ref[...], wt_ref[...], preferred_element_type=jnp.float32) + b_ref[...]
    t0_ref[...] = ht[:, 0:128]
    t1_ref[...] = ht[:, 128:256]
    t2_ref[...] = ht[:, 256:384]
    t3_ref[...] = ht[:, 384:512]
    n = nodes_ref[...]
    rows, K = n.shape
    dup = jnp.zeros(n.shape, jnp.bool_)
    for d in range(1, K):
        shifted = jnp.concatenate(
            [jnp.full((rows, d), -1, jnp.int32), n[:, : K - d]], axis=1)
        dup = jnp.logical_or(dup, n == shifted)
    cc_ref[...] = 1.0 - dup.astype(jnp.float32)


def _node_kernel(x_ref, wt_ref, b_ref, ce_ref, s_ref):
    t = jnp.dot(x_ref[...], wt_ref[...], preferred_element_type=jnp.float32) + b_ref[...]
    lr = jnp.where(t >= 0, t, 0.01 * t)
    s_ref[...] = jnp.dot(lr, ce_ref[...], preferred_element_type=jnp.float32)


def _combine_kernel(H, S_ref, s_ref, cnt_ref, o_ref):
    s = s_ref[...]
    c = cnt_ref[...]
    m = jnp.maximum(s, 0.0)
    e = jnp.maximum(jnp.exp(s - m), 1e-35)
    denom = c * e + (float(H) - c) * jnp.exp(-m)
    a = e / denom
    o_ref[...] = jnp.maximum(a * S_ref[0], 0.0)


def _make_sc_scatter(NPAD, n_tiles, blocks_per_tile, stripe):
    mesh = plsc.VectorSubcoreMesh(core_axis_name="c", subcore_axis_name="s")

    @functools.partial(
        pl.kernel,
        mesh=mesh,
        out_type=[
            jax.ShapeDtypeStruct((4, NPAD, 128), jnp.float32),
            jax.ShapeDtypeStruct((NPAD,), jnp.float32),
        ],
        scratch_types=[
            pltpu.VMEM((blocks_per_tile, 128), jnp.int32),
            pltpu.VMEM((blocks_per_tile, 128), jnp.int32),
            pltpu.VMEM((blocks_per_tile, 128), jnp.float32),
            pltpu.VMEM((128, 128), jnp.float32),
            pltpu.VMEM((128, 128), jnp.float32),
            pltpu.VMEM_SHARED((NPAD, 128), jnp.float32),
            pltpu.VMEM_SHARED((NPAD,), jnp.float32),
            pltpu.SemaphoreType.DMA,
            pltpu.SemaphoreType.DMA,
            pltpu.SemaphoreType.DMA,
            pltpu.SemaphoreType.DMA,
        ],
    )
    def sc_scatter(nodes3, eids3, cntv3, zrows, z1, t0, t1, t2, t3,
                   S_out, cnt_out,
                   idxn_v, idxe_v, cval_v, rows_a, rows_b, S_sh, cnt_sh,
                   ga, gb, sa, sb):
        cid = lax.axis_index("c")
        sid = lax.axis_index("s")

        pltpu.sync_copy(nodes3.at[sid], idxn_v)
        pltpu.sync_copy(eids3.at[sid], idxe_v)

        @pl.when(cid == 0)
        def _():
            pltpu.sync_copy(z1, cnt_sh.at[pl.ds(sid * stripe, stripe)])
            plsc.subcore_barrier()
            pltpu.sync_copy(cntv3.at[sid], cval_v)

            def cbody(j, carry):
                pltpu.sync_copy(cval_v.at[j], cnt_sh.at[idxn_v.at[j]], add=True)
                return carry

            lax.fori_loop(0, blocks_per_tile, cbody, 0)
            plsc.subcore_barrier()
            pltpu.sync_copy(cnt_sh.at[pl.ds(sid * stripe, stripe)],
                            cnt_out.at[pl.ds(sid * stripe, stripe)])

        half = blocks_per_tile // 2
        for ti, table in enumerate((t0, t1, t2, t3)):
            @pl.when(cid == ti // 2)
            def _(table=table, ti=ti):
                pltpu.sync_copy(zrows, S_sh.at[pl.ds(sid * stripe, stripe)])
                plsc.subcore_barrier()

                pltpu.async_copy(table.at[idxe_v.at[0]], rows_a, ga)
                pltpu.async_copy(table.at[idxe_v.at[1]], rows_b, gb)

                def body(j, carry):
                    b0 = 2 * j
                    b1 = 2 * j + 1
                    pltpu.make_async_copy(table.at[idxe_v.at[b0]], rows_a, ga).wait()
                    pltpu.async_copy(rows_a, S_sh.at[idxn_v.at[b0]], sa, add=True)
                    pltpu.make_async_copy(table.at[idxe_v.at[b1]], rows_b, gb).wait()
                    pltpu.async_copy(rows_b, S_sh.at[idxn_v.at[b1]], sb, add=True)

                    @pl.when(j < half - 1)
                    def _():
                        pltpu.make_async_copy(rows_a, S_sh.at[idxn_v.at[b0]], sa).wait()
                        pltpu.async_copy(table.at[idxe_v.at[b0 + 2]], rows_a, ga)
                        pltpu.make_async_copy(rows_b, S_sh.at[idxn_v.at[b1]], sb).wait()
                        pltpu.async_copy(table.at[idxe_v.at[b1 + 2]], rows_b, gb)

                    return carry

                lax.fori_loop(0, half, body, 0)
                pltpu.make_async_copy(
                    rows_a, S_sh.at[idxn_v.at[blocks_per_tile - 2]], sa).wait()
                pltpu.make_async_copy(
                    rows_b, S_sh.at[idxn_v.at[blocks_per_tile - 1]], sb).wait()
                plsc.subcore_barrier()
                pltpu.sync_copy(S_sh.at[pl.ds(sid * stripe, stripe)],
                                S_out.at[ti, pl.ds(sid * stripe, stripe)])

    return sc_scatter


def kernel(node_embeddings, hyperedge_embeddings, hyperedge_to_nodes, W_fc, b_fc, W_we, b_we, c_e):
    N, d_in = node_embeddings.shape
    H, K = hyperedge_to_nodes.shape
    d_out = W_fc.shape[0]
    assert d_in == 512 and d_out == 512

    n_tiles = 16
    stripe = 640
    NPAD = n_tiles * stripe
    PAIRS = H * K
    pairs_per_tile = PAIRS // n_tiles
    blocks_per_tile = pairs_per_tile // 128

    f32 = jnp.float32

    eb = 256
    ht0, ht1, ht2, ht3, cc = pl.pallas_call(
        _edge_kernel,
        grid=(H // eb,),
        in_specs=[
            pl.BlockSpec((eb, d_in), lambda i: (i, 0)),
            pl.BlockSpec((d_in, d_out), lambda i: (0, 0)),
            pl.BlockSpec((1, d_out), lambda i: (0, 0)),
            pl.BlockSpec((eb, K), lambda i: (i, 0)),
        ],
        out_specs=[pl.BlockSpec((eb, 128), lambda i: (i, 0))] * 4
        + [pl.BlockSpec((eb, K), lambda i: (i, 0))],
        out_shape=[jax.ShapeDtypeStruct((H, 128), f32)] * 4
        + [jax.ShapeDtypeStruct((H, K), f32)],
    )(hyperedge_embeddings, W_fc.T, b_fc.reshape(1, -1), hyperedge_to_nodes)

    nb = 640
    ne_pad = jnp.pad(node_embeddings, ((0, NPAD - N), (0, 0)))
    s_col = pl.pallas_call(
        _node_kernel,
        grid=(NPAD // nb,),
        in_specs=[
            pl.BlockSpec((nb, d_in), lambda i: (i, 0)),
            pl.BlockSpec((d_in, d_out), lambda i: (0, 0)),
            pl.BlockSpec((1, d_out), lambda i: (0, 0)),
            pl.BlockSpec((d_out, 1), lambda i: (0, 0)),
        ],
        out_specs=pl.BlockSpec((nb, 1), lambda i: (i, 0)),
        out_shape=jax.ShapeDtypeStruct((NPAD, 1), f32),
    )(ne_pad, W_we.T, b_we.reshape(1, -1), c_e.reshape(-1, 1))

    nodes3 = hyperedge_to_nodes.reshape(n_tiles, blocks_per_tile, 128)
    eids3 = (jnp.arange(PAIRS, dtype=jnp.int32) // K).reshape(
        n_tiles, blocks_per_tile, 128)
    cntv3 = cc.reshape(n_tiles, blocks_per_tile, 128)
    zrows = jnp.zeros((stripe, 128), f32)
    z1 = jnp.zeros((stripe,), f32)

    sc = _make_sc_scatter(NPAD, n_tiles, blocks_per_tile, stripe)
    S_chunks, cnt = sc(nodes3, eids3, cntv3, zrows, z1, ht0, ht1, ht2, ht3)

    out_pad = pl.pallas_call(
        functools.partial(_combine_kernel, H),
        grid=(NPAD // nb, 4),
        in_specs=[
            pl.BlockSpec((1, nb, 128), lambda r, c: (c, r, 0)),
            pl.BlockSpec((nb, 1), lambda r, c: (r, 0)),
            pl.BlockSpec((nb, 1), lambda r, c: (r, 0)),
        ],
        out_specs=pl.BlockSpec((nb, 128), lambda r, c: (r, c)),
        out_shape=jax.ShapeDtypeStruct((NPAD, d_out), f32),
    )(S_chunks, s_col, cnt.reshape(NPAD, 1))

    return out_pad[:N]


# speedup vs baseline: 5.2936x; 1.0175x over previous
import functools

import jax
import jax.numpy as jnp
from jax import lax
from jax.experimental import pallas as pl
from jax.experimental.pallas import tpu as pltpu
from jax.experimental.pallas import tpu_sc as plsc


def _edge_kernel(he_ref, wt_ref, b_ref, nodes_ref, t0_ref, t1_ref, t2_ref, t3_ref, cc_ref):
    ht = jnp.dot(he_ref[...], wt_ref[...], preferred_element_type=jnp.float32) + b_ref[...]
    t0_ref[...] = ht[:, 0:128]
    t1_ref[...] = ht[:, 128:256]
    t2_ref[...] = ht[:, 256:384]
    t3_ref[...] = ht[:, 384:512]
    n = nodes_ref[...]
    rows, K = n.shape
    dup = jnp.zeros(n.shape, jnp.bool_)
    for d in range(1, K):
        shifted = jnp.concatenate(
            [jnp.full((rows, d), -1, jnp.int32), n[:, : K - d]], axis=1)
        dup = jnp.logical_or(dup, n == shifted)
    cc_ref[...] = 1.0 - dup.astype(jnp.float32)


def _node_kernel(x_ref, wt_ref, b_ref, ce_ref, s_ref):
    t = jnp.dot(x_ref[...], wt_ref[...], preferred_element_type=jnp.float32) + b_ref[...]
    lr = jnp.where(t >= 0, t, 0.01 * t)
    s_ref[...] = jnp.dot(lr, ce_ref[...], preferred_element_type=jnp.float32)


def _combine_kernel(H, S_ref, s_ref, cnt_ref, o_ref):
    s = s_ref[...]
    c = cnt_ref[...]
    m = jnp.maximum(s, 0.0)
    e = jnp.maximum(jnp.exp(s - m), 1e-35)
    denom = c * e + (float(H) - c) * jnp.exp(-m)
    a = e / denom
    o_ref[...] = jnp.maximum(a * S_ref[0], 0.0)


def _make_sc_scatter(NPAD, n_tiles, blocks_per_tile, stripe):
    mesh = plsc.VectorSubcoreMesh(core_axis_name="c", subcore_axis_name="s")

    @functools.partial(
        pl.kernel,
        mesh=mesh,
        out_type=[
            jax.ShapeDtypeStruct((4, NPAD, 128), jnp.float32),
            jax.ShapeDtypeStruct((NPAD,), jnp.float32),
        ],
        scratch_types=[
            pltpu.VMEM((blocks_per_tile, 128), jnp.int32),
            pltpu.VMEM((blocks_per_tile, 128), jnp.int32),
            pltpu.VMEM((blocks_per_tile, 128), jnp.float32),
            pltpu.VMEM((128, 128), jnp.float32),
            pltpu.VMEM((128, 128), jnp.float32),
            pltpu.VMEM_SHARED((NPAD, 128), jnp.float32),
            pltpu.VMEM_SHARED((NPAD,), jnp.float32),
            pltpu.SemaphoreType.DMA,
            pltpu.SemaphoreType.DMA,
            pltpu.SemaphoreType.DMA,
            pltpu.SemaphoreType.DMA,
        ],
    )
    def sc_scatter(nodes3, eids3, cntv3, zrows, z1, t0, t1, t2, t3,
                   S_out, cnt_out,
                   idxn_v, idxe_v, cval_v, rows_a, rows_b, S_sh, cnt_sh,
                   ga, gb, sa, sb):
        cid = lax.axis_index("c")
        sid = lax.axis_index("s")

        pltpu.sync_copy(nodes3.at[sid], idxn_v)
        pltpu.sync_copy(eids3.at[sid], idxe_v)

        @pl.when(cid == 0)
        def _():
            pltpu.sync_copy(z1, cnt_sh.at[pl.ds(sid * stripe, stripe)])
            plsc.subcore_barrier()
            pltpu.sync_copy(cntv3.at[sid], cval_v)

            def cbody(j, carry):
                pltpu.sync_copy(cval_v.at[j], cnt_sh.at[idxn_v.at[j]], add=True)
                return carry

            lax.fori_loop(0, blocks_per_tile, cbody, 0)
            plsc.subcore_barrier()
            pltpu.sync_copy(cnt_sh.at[pl.ds(sid * stripe, stripe)],
                            cnt_out.at[pl.ds(sid * stripe, stripe)])

        half = blocks_per_tile // 2
        for ti, table in enumerate((t0, t1, t2, t3)):
            @pl.when(cid == ti // 2)
            def _(table=table, ti=ti):
                pltpu.sync_copy(zrows, S_sh.at[pl.ds(sid * stripe, stripe)])
                plsc.subcore_barrier()

                pltpu.async_copy(table.at[idxe_v.at[0]], rows_a, ga)
                pltpu.async_copy(table.at[idxe_v.at[1]], rows_b, gb)

                def body(j, carry):
                    b0 = 2 * j
                    b1 = 2 * j + 1
                    pltpu.make_async_copy(table.at[idxe_v.at[b0]], rows_a, ga).wait()
                    pltpu.async_copy(rows_a, S_sh.at[idxn_v.at[b0]], sa, add=True)
                    pltpu.make_async_copy(table.at[idxe_v.at[b1]], rows_b, gb).wait()
                    pltpu.async_copy(rows_b, S_sh.at[idxn_v.at[b1]], sb, add=True)

                    @pl.when(j < half - 1)
                    def _():
                        pltpu.make_async_copy(rows_a, S_sh.at[idxn_v.at[b0]], sa).wait()
                        pltpu.async_copy(table.at[idxe_v.at[b0 + 2]], rows_a, ga)
                        pltpu.make_async_copy(rows_b, S_sh.at[idxn_v.at[b1]], sb).wait()
                        pltpu.async_copy(table.at[idxe_v.at[b1 + 2]], rows_b, gb)

                    return carry

                lax.fori_loop(0, half, body, 0)
                pltpu.make_async_copy(
                    rows_a, S_sh.at[idxn_v.at[blocks_per_tile - 2]], sa).wait()
                pltpu.make_async_copy(
                    rows_b, S_sh.at[idxn_v.at[blocks_per_tile - 1]], sb).wait()
                plsc.subcore_barrier()
                pltpu.sync_copy(S_sh.at[pl.ds(sid * stripe, stripe)],
                                S_out.at[ti, pl.ds(sid * stripe, stripe)])

    return sc_scatter


def kernel(node_embeddings, hyperedge_embeddings, hyperedge_to_nodes, W_fc, b_fc, W_we, b_we, c_e):
    N, d_in = node_embeddings.shape
    H, K = hyperedge_to_nodes.shape
    d_out = W_fc.shape[0]
    assert d_in == 512 and d_out == 512

    n_tiles = 16
    stripe = 640
    NPAD = n_tiles * stripe
    PAIRS = H * K
    pairs_per_tile = PAIRS // n_tiles
    blocks_per_tile = pairs_per_tile // 128

    f32 = jnp.float32

    eb = 256
    ht0, ht1, ht2, ht3, cc = pl.pallas_call(
        _edge_kernel,
        grid=(H // eb,),
        in_specs=[
            pl.BlockSpec((eb, d_in), lambda i: (i, 0)),
            pl.BlockSpec((d_in, d_out), lambda i: (0, 0)),
            pl.BlockSpec((1, d_out), lambda i: (0, 0)),
            pl.BlockSpec((eb, K), lambda i: (i, 0)),
        ],
        out_specs=[pl.BlockSpec((eb, 128), lambda i: (i, 0))] * 4
        + [pl.BlockSpec((eb, K), lambda i: (i, 0))],
        out_shape=[jax.ShapeDtypeStruct((H, 128), f32)] * 4
        + [jax.ShapeDtypeStruct((H, K), f32)],
    )(hyperedge_embeddings, W_fc.T, b_fc.reshape(1, -1), hyperedge_to_nodes)

    nb = 400
    s_col = pl.pallas_call(
        _node_kernel,
        grid=(N // nb,),
        in_specs=[
            pl.BlockSpec((nb, d_in), lambda i: (i, 0)),
            pl.BlockSpec((d_in, d_out), lambda i: (0, 0)),
            pl.BlockSpec((1, d_out), lambda i: (0, 0)),
            pl.BlockSpec((d_out, 1), lambda i: (0, 0)),
        ],
        out_specs=pl.BlockSpec((nb, 1), lambda i: (i, 0)),
        out_shape=jax.ShapeDtypeStruct((N, 1), f32),
    )(node_embeddings, W_we.T, b_we.reshape(1, -1), c_e.reshape(-1, 1))

    nodes3 = hyperedge_to_nodes.reshape(n_tiles, blocks_per_tile, 128)
    eids3 = (jnp.arange(PAIRS, dtype=jnp.int32) // K).reshape(
        n_tiles, blocks_per_tile, 128)
    cntv3 = cc.reshape(n_tiles, blocks_per_tile, 128)
    zrows = jnp.zeros((stripe, 128), f32)
    z1 = jnp.zeros((stripe,), f32)

    sc = _make_sc_scatter(NPAD, n_tiles, blocks_per_tile, stripe)
    S_chunks, cnt = sc(nodes3, eids3, cntv3, zrows, z1, ht0, ht1, ht2, ht3)

    out = pl.pallas_call(
        functools.partial(_combine_kernel, H),
        grid=(N // nb, 4),
        in_specs=[
            pl.BlockSpec((1, nb, 128), lambda r, c: (c, r, 0)),
            pl.BlockSpec((nb, 1), lambda r, c: (r, 0)),
            pl.BlockSpec((nb, 1), lambda r, c: (r, 0)),
        ],
        out_specs=pl.BlockSpec((nb, 128), lambda r, c: (r, c)),
        out_shape=jax.ShapeDtypeStruct((N, d_out), f32),
    )(S_chunks, s_col, cnt[:N].reshape(N, 1))

    return out
